# baseline (speedup 1.0000x reference)
import functools

import jax
import jax.numpy as jnp
import numpy as np
from jax import lax
from jax.experimental import pallas as pl
from jax.experimental.pallas import tpu as pltpu
from jax.experimental.pallas import tpu_sc as plsc

_N = 10000
_E = 320000
_NP = 10240
_NP4 = _NP * 4
_CB = 128
_NCHUNK = 160
_EPT = _NCHUNK * _CB
_EPAD = 16 * _EPT
_LAYER_DIMS = [(192, 8, 32), (256, 8, 32), (256, 8, 32), (256, 8, 24)]

_NEG = -1.0e30


def _f16(v):
    return jnp.full((16,), v, jnp.int32)


@functools.lru_cache(maxsize=None)
def _edge_kernel(D, want_alpha):
    Wc = 2 * D
    _NP2 = _NP * 2
    inv_sqrt_d = float(1.0 / np.sqrt(D))
    mesh = plsc.VectorSubcoreMesh(core_axis_name="c", subcore_axis_name="s")

    out_type = [
        jax.ShapeDtypeStruct((2, 2, _NP, Wc), jnp.float32),
        jax.ShapeDtypeStruct((2, 16, _NP2), jnp.float32),
        jax.ShapeDtypeStruct((2, _NP2), jnp.float32),
        jax.ShapeDtypeStruct((2, 16, _NCHUNK, 2, _CB), jnp.float32),
        jax.ShapeDtypeStruct((2, _NP2), jnp.float32),
    ]
    if want_alpha:
        out_type.append(
            jax.ShapeDtypeStruct((2, 2, 16, _NCHUNK, 2, _CB), jnp.float32))

    scratch_types = [
        pltpu.VMEM((_CB,), jnp.int32),
        pltpu.VMEM((_CB,), jnp.int32),
        pltpu.VMEM((_NP2,), jnp.float32),
        pltpu.VMEM((_CB, Wc), jnp.float32),
        pltpu.VMEM((_CB, Wc), jnp.float32),
        pltpu.VMEM((_CB, 8), jnp.float32),
        pltpu.VMEM((2, _CB), jnp.float32),
        pltpu.VMEM((_CB,), jnp.int32),
        pltpu.VMEM((_CB,), jnp.int32),
        pltpu.VMEM((1, _CB), jnp.int32),
        pltpu.VMEM((2, 1280), jnp.float32),
        pltpu.VMEM((128,), jnp.float32),
        pltpu.VMEM_SHARED((_NP, Wc), jnp.float32),
        pltpu.VMEM_SHARED((_NP, 8), jnp.float32),
        pltpu.SemaphoreType.DMA,
        pltpu.SemaphoreType.DMA,
    ]

    def body(qf, kf, vf, srcf, dstf,
             agg_o, mstage_o, mmerged_o, exl_o, den2_o, *rest):
        if want_alpha:
            alpha_o = rest[0]
            rest = rest[1:]
        (srcb, dstb, m_loc, qbuf, kbuf, exbuf, lbuf, gq, gk, sidx,
         mbuf, dbuf, aggsh, densh, sem1, sem2) = rest
        cc = lax.axis_index("c")
        ss = lax.axis_index("s")
        r0 = ss * (_NP // 16)
        z16 = jnp.zeros((16,), jnp.float32)

        def _round(rnd, rcarry):
            qoff = (cc * 2 + rnd) * _NP

            def _init_m(i, carry):
                m_loc[pl.ds(i * 16, 16)] = jnp.full((16,), _NEG, jnp.float32)
                return carry
            lax.fori_loop(0, _NP2 // 16, _init_m, 0)

            for r in range(64):
                for w in range(Wc // 16):
                    kbuf[r, pl.ds(w * 16, 16)] = z16
            for i in range(_CB * 8 // 16):
                fl = lax.iota(jnp.int32, 16) + i * 16
                plsc.store_scatter(exbuf, [fl // 8, fl % 8], z16)
            for b in range(_NP // 16 // 64):
                pltpu.sync_copy(kbuf.at[pl.ds(0, 64)],
                                aggsh.at[pl.ds(r0 + b * 64, 64)])
                pltpu.sync_copy(exbuf.at[pl.ds(0, 64)],
                                densh.at[pl.ds(r0 + b * 64, 64)])

            def _pass_a(j, carry):
                base = (ss * _NCHUNK + j) * _CB
                pltpu.sync_copy(srcf.at[pl.ds(base, _CB)], srcb)
                pltpu.sync_copy(dstf.at[pl.ds(base, _CB)], dstb)
                for i in range(_CB // 16):
                    sl = pl.ds(i * 16, 16)
                    gq[sl] = dstb[sl] + qoff
                    gk[sl] = srcb[sl] + qoff
                cp1 = pltpu.async_copy(qf.at[gq], qbuf, sem1)
                cp2 = pltpu.async_copy(kf.at[gk], kbuf, sem2)
                cp1.wait()
                cp2.wait()
                for g in range(_CB // 16):
                    sl = pl.ds(g * 16, 16)
                    ev = lax.iota(jnp.int32, 16) + g * 16
                    dv = dstb[sl]
                    eid = base + g * 16 + lax.iota(jnp.int32, 16)
                    valid = eid < _E
                    for h in range(2):
                        acc = jnp.zeros((16,), jnp.float32)
                        for d in range(D):
                            col = _f16(h * D + d)
                            acc = acc + (plsc.load_gather(qbuf, [ev, col]) *
                                         plsc.load_gather(kbuf, [ev, col]))
                        lg = jnp.where(valid, acc * inv_sqrt_d, _NEG)
                        lbuf[h, sl] = lg
                        didx = dv * 2 + h
                        cur = plsc.load_gather(m_loc, [didx])
                        need = lg > cur

                        def _mcond(nd):
                            return plsc.all_reduce_population_count(nd)[0] > 0

                        def _mbody(nd):
                            plsc.store_scatter(m_loc, [didx], lg, mask=nd)
                            c2 = plsc.load_gather(m_loc, [didx])
                            return jnp.logical_and(nd, lg > c2)

                        lax.while_loop(_mcond, _mbody, need)
                pltpu.sync_copy(lbuf, exl_o.at[cc, ss, j])
                return carry
            lax.fori_loop(0, _NCHUNK, _pass_a, 0)

            pltpu.sync_copy(m_loc, mstage_o.at[cc, ss])
            plsc.subcore_barrier()
            ms = ss * 1280
            pltpu.sync_copy(mstage_o.at[cc, 0, pl.ds(ms, 1280)], mbuf.at[0])
            for t in range(1, 16):
                pltpu.sync_copy(mstage_o.at[cc, t, pl.ds(ms, 1280)], mbuf.at[1])

                def _mmax(i, carry):
                    sl = pl.ds(i * 16, 16)
                    mbuf[0, sl] = jnp.maximum(mbuf[0, sl], mbuf[1, sl])
                    return carry
                lax.fori_loop(0, 80, _mmax, 0)
            pltpu.sync_copy(mbuf.at[0], mmerged_o.at[cc, pl.ds(ms, 1280)])
            plsc.subcore_barrier()
            pltpu.sync_copy(mmerged_o.at[cc], m_loc)

            def _pass_b(j, carry):
                base = (ss * _NCHUNK + j) * _CB
                pltpu.sync_copy(exl_o.at[cc, ss, j], lbuf)
                pltpu.sync_copy(srcf.at[pl.ds(base, _CB)], srcb)
                pltpu.sync_copy(dstf.at[pl.ds(base, _CB)], dstb)
                for i in range(_CB // 16):
                    sl = pl.ds(i * 16, 16)
                    gk[sl] = srcb[sl] + qoff
                    sidx[0, sl] = dstb[sl]
                pltpu.async_copy(vf.at[gk], qbuf, sem1).wait()
                for g in range(_CB // 16):
                    sl = pl.ds(g * 16, 16)
                    ev = lax.iota(jnp.int32, 16) + g * 16
                    dv = dstb[sl]
                    eid = base + g * 16 + lax.iota(jnp.int32, 16)
                    valid = eid < _E
                    for h in range(2):
                        lg = lbuf[h, sl]
                        mg = plsc.load_gather(m_loc, [dv * 2 + h])
                        ex = jnp.where(valid, jnp.exp(lg - mg), 0.0)
                        lbuf[h, sl] = ex
                        plsc.store_scatter(exbuf, [ev, _f16(h)], ex)
                        for d in range(D):
                            col = _f16(h * D + d)
                            vv = plsc.load_gather(qbuf, [ev, col])
                            plsc.store_scatter(qbuf, [ev, col], vv * ex)
                if want_alpha:
                    pltpu.sync_copy(lbuf, exl_o.at[cc, ss, j])
                pltpu.sync_copy(qbuf, aggsh.at[sidx.at[0]], add=True)
                pltpu.sync_copy(exbuf, densh.at[sidx.at[0]], add=True)
                return carry
            lax.fori_loop(0, _NCHUNK, _pass_b, 0)
            plsc.subcore_barrier()

            def _pass_d(b, carry):
                r = r0 + b * 64
                pltpu.sync_copy(aggsh.at[pl.ds(r, 64)], qbuf.at[pl.ds(0, 64)])
                pltpu.sync_copy(densh.at[pl.ds(r, 64)], exbuf.at[pl.ds(0, 64)])
                for g in range(4):
                    nv = lax.iota(jnp.int32, 16) + g * 16
                    for h in range(2):
                        dv = plsc.load_gather(exbuf, [nv, _f16(h)])
                        plsc.store_scatter(dbuf, [nv * 2 + h], dv)
                        rec = 1.0 / (dv + 1e-16)
                        for d in range(D):
                            col = _f16(h * D + d)
                            av = plsc.load_gather(qbuf, [nv, col])
                            plsc.store_scatter(qbuf, [nv, col], av * rec)
                pltpu.sync_copy(qbuf.at[pl.ds(0, 64)],
                                agg_o.at[cc, rnd, pl.ds(r, 64)])
                pltpu.sync_copy(dbuf, den2_o.at[cc, pl.ds(r * 2, 128)])
                return carry
            lax.fori_loop(0, _NP // 16 // 64, _pass_d, 0)

            if want_alpha:
                plsc.subcore_barrier()
                pltpu.sync_copy(den2_o.at[cc], m_loc)

                def _pass_c(j, carry):
                    base = (ss * _NCHUNK + j) * _CB
                    pltpu.sync_copy(exl_o.at[cc, ss, j], lbuf)
                    pltpu.sync_copy(dstf.at[pl.ds(base, _CB)], dstb)
                    for g in range(_CB // 16):
                        sl = pl.ds(g * 16, 16)
                        dv = dstb[sl]
                        for h in range(2):
                            ex = lbuf[h, sl]
                            dg = plsc.load_gather(m_loc, [dv * 2 + h])
                            lbuf[h, sl] = ex / (dg + 1e-16)
                    pltpu.sync_copy(lbuf, alpha_o.at[cc, rnd, ss, j])
                    return carry
                lax.fori_loop(0, _NCHUNK, _pass_c, 0)
            plsc.subcore_barrier()
            return rcarry
        lax.fori_loop(0, 2, _round, 0)

    return pl.kernel(body, out_type=tuple(out_type), mesh=mesh,
                     scratch_types=tuple(scratch_types),
                     compiler_params=pltpu.CompilerParams(
                         needs_layout_passes=False, use_tc_tiling_on_sc=False))


def _gelu(x):
    return 0.5 * x * (1.0 + lax.erf(x * np.float32(0.7071067811865476)))

_RB = 256
_NB = _NP // _RB


@functools.lru_cache(maxsize=None)
def _time_kernel():
    mesh = plsc.VectorSubcoreMesh(core_axis_name="c", subcore_axis_name="s")
    scratch_types = (
        pltpu.VMEM((64,), jnp.int32),
        pltpu.VMEM((64, 32), jnp.float32),
        pltpu.SemaphoreType.DMA,
    )

    def body(time_ref, temb_ref, out_ref, idxb, rows, sem):
        wid = lax.axis_index("s") * 2 + lax.axis_index("c")

        def _chunk(i, carry):
            base = wid * (_NP // 32) + i * 64
            pltpu.sync_copy(time_ref.at[pl.ds(base, 64)], idxb)
            pltpu.async_copy(temb_ref.at[idxb], rows, sem).wait()
            pltpu.sync_copy(rows, out_ref.at[pl.ds(base, 64)])
            return carry
        lax.fori_loop(0, _NP // 32 // 64, _chunk, 0)

    return pl.kernel(body, out_type=jax.ShapeDtypeStruct((_NP, 32), jnp.float32),
                     mesh=mesh, scratch_types=scratch_types,
                     compiler_params=pltpu.CompilerParams(
                         needs_layout_passes=False, use_tc_tiling_on_sc=False))


def _pre_kernel(pcd8, xy8, tf, p):
    def body(pcd_ref, xy_ref, tf_ref, w1, b1, w2, b2, w3, b3,
             pw1, pb1, pw2, pb2, mw1, mb1, mw2, mb2, o_ref):
        f32 = jnp.float32
        hh = jnp.maximum(jnp.dot(pcd_ref[...], w1[...], preferred_element_type=f32) + b1[...], 0.0)
        hh = jnp.maximum(jnp.dot(hh, w2[...], preferred_element_type=f32) + b2[...], 0.0)
        hh = jnp.dot(hh, w3[...], preferred_element_type=f32) + b3[...]
        pf = jnp.max(hh.reshape(_RB, 20, 128), axis=1)
        g1 = _gelu(jnp.dot(xy_ref[...], pw1[...], preferred_element_type=f32) + pb1[...])
        pos = jnp.dot(g1, pw2[...], preferred_element_type=f32) + pb2[...]
        comb = jnp.concatenate([pf, pos, tf_ref[...]], axis=-1)
        c1 = jnp.dot(comb, mw1[...], preferred_element_type=f32) + mb1[...]
        c1 = jnp.where(c1 > 0, c1, 0.2 * c1)
        c2 = jnp.dot(c1, mw2[...], preferred_element_type=f32) + mb2[...]
        o_ref[...] = jnp.where(c2 > 0, c2, 0.2 * c2)

    def _w(a):
        return pl.BlockSpec(a.shape, lambda i: (0,) * a.ndim)

    w1 = jnp.pad(p['pn_W1'], ((0, 5), (0, 0)))
    pw1 = jnp.pad(p['pos_W1'], ((0, 1), (0, 0)))
    ws = [w1, p['pn_b1'].reshape(1, -1), p['pn_W2'], p['pn_b2'].reshape(1, -1),
          p['pn_W3'], p['pn_b3'].reshape(1, -1), pw1, p['pos_b1'].reshape(1, -1),
          p['pos_W2'], p['pos_b2'].reshape(1, -1), p['mlp_W1'], p['mlp_b1'].reshape(1, -1),
          p['mlp_W2'], p['mlp_b2'].reshape(1, -1)]
    return pl.pallas_call(
        body, grid=(_NB,),
        in_specs=[pl.BlockSpec((_RB * 20, 8), lambda i: (i, 0)),
                  pl.BlockSpec((_RB, 8), lambda i: (i, 0)),
                  pl.BlockSpec((_RB, 32), lambda i: (i, 0))] + [_w(a) for a in ws],
        out_specs=pl.BlockSpec((_RB, 192), lambda i: (i, 0)),
        out_shape=jax.ShapeDtypeStruct((_NP, 192), jnp.float32),
    )(pcd8, xy8, tf, *ws)


def _proj(x, w, D):
    Wc = 2 * D
    din = x.shape[1]
    w4 = w.reshape(din, 4, Wc).transpose(1, 0, 2)

    def body(x_ref, w_ref, o_ref):
        o_ref[...] = jnp.dot(x_ref[...], w_ref[0], preferred_element_type=jnp.float32)

    return pl.pallas_call(
        body, grid=(4, _NB),
        in_specs=[pl.BlockSpec((_RB, din), lambda q, i: (i, 0)),
                  pl.BlockSpec((1, din, Wc), lambda q, i: (q, 0, 0))],
        out_specs=pl.BlockSpec((_RB, Wc), lambda q, i: (q * _NB + i, 0)),
        out_shape=jax.ShapeDtypeStruct((4 * _NP, Wc), jnp.float32),
    )(x, w4)


def _mm(x, w):
    din = x.shape[1]
    dout = w.shape[1]

    def body(x_ref, w_ref, o_ref):
        o_ref[...] = jnp.dot(x_ref[...], w_ref[...], preferred_element_type=jnp.float32)

    return pl.pallas_call(
        body, grid=(_NB,),
        in_specs=[pl.BlockSpec((_RB, din), lambda i: (i, 0)),
                  pl.BlockSpec((din, dout), lambda i: (0, 0))],
        out_specs=pl.BlockSpec((_RB, dout), lambda i: (i, 0)),
        out_shape=jax.ShapeDtypeStruct((_NP, dout), jnp.float32),
    )(x, w)


def _combine(agg4, skip, act):
    Wc = agg4.shape[-1]

    def body(a_ref, s_ref, o_ref):
        a = a_ref[...]
        x = jnp.concatenate([a[0], a[1], a[2], a[3]], axis=-1) + s_ref[...]
        o_ref[...] = jnp.maximum(x, 0.0) if act else x

    return pl.pallas_call(
        body, grid=(_NB,),
        in_specs=[pl.BlockSpec((4, _RB, Wc), lambda i: (0, i, 0)),
                  pl.BlockSpec((_RB, 4 * Wc), lambda i: (i, 0))],
        out_specs=pl.BlockSpec((_RB, 4 * Wc), lambda i: (i, 0)),
        out_shape=jax.ShapeDtypeStruct((_NP, 4 * Wc), jnp.float32),
    )(agg4, skip)


def _heads_kernel(feats, comb, p):
    def body(f_ref, c_ref, tw1, tb1, tw2, tb2, rw1, rb1, rw2, rb2, o_ref):
        f32 = jnp.float32
        y = f_ref[...] + c_ref[...]
        ht = _gelu(jnp.dot(y, tw1[...], preferred_element_type=f32) + tb1[...])
        tp = jnp.dot(ht, tw2[...], preferred_element_type=f32) + tb2[...]
        hr = _gelu(jnp.dot(y, rw1[...], preferred_element_type=f32) + rb1[...])
        rv = jnp.dot(hr, rw2[...], preferred_element_type=f32) + rb2[...]
        vx = rv[:, 0:1]; vy = rv[:, 1:2]; vz = rv[:, 2:3]
        t2 = vx * vx + vy * vy + vz * vz
        th = jnp.sqrt(t2 + 1e-20)
        small = th < 1e-4
        st = jnp.where(small, 1.0, th)
        st2 = jnp.where(small, 1.0, t2)
        a = jnp.where(small, 1.0 - t2 / 6.0, jnp.sin(th) / st)
        b = jnp.where(small, 0.5 - t2 / 24.0, (1.0 - jnp.cos(th)) / st2)
        m00 = 1.0 - b * (t2 - vx * vx); m11 = 1.0 - b * (t2 - vy * vy)
        m22 = 1.0 - b * (t2 - vz * vz)
        m01 = b * vx * vy - a * vz; m10 = b * vx * vy + a * vz
        m02 = b * vx * vz + a * vy; m20 = b * vx * vz - a * vy
        m12 = b * vy * vz - a * vx; m21 = b * vy * vz + a * vx

        def sqp(x):
            return jnp.where(x > 0, jnp.sqrt(jnp.maximum(x, 1e-20)), 0.0)

        qa0 = sqp(1.0 + m00 + m11 + m22); qa1 = sqp(1.0 + m00 - m11 - m22)
        qa2 = sqp(1.0 - m00 + m11 - m22); qa3 = sqp(1.0 - m00 - m11 + m22)
        d0 = 2.0 * jnp.maximum(qa0, 0.1); d1 = 2.0 * jnp.maximum(qa1, 0.1)
        d2 = 2.0 * jnp.maximum(qa2, 0.1); d3 = 2.0 * jnp.maximum(qa3, 0.1)
        cand = [
            [qa0 * qa0 / d0, (m21 - m12) / d0, (m02 - m20) / d0, (m10 - m01) / d0],
            [(m21 - m12) / d1, qa1 * qa1 / d1, (m10 + m01) / d1, (m02 + m20) / d1],
            [(m02 - m20) / d2, (m10 + m01) / d2, qa2 * qa2 / d2, (m12 + m21) / d2],
            [(m10 - m01) / d3, (m20 + m02) / d3, (m21 + m12) / d3, qa3 * qa3 / d3],
        ]
        p0 = (qa0 >= qa1) & (qa0 >= qa2) & (qa0 >= qa3)
        p1 = (qa1 >= qa2) & (qa1 >= qa3)
        p2 = qa2 >= qa3
        q = [jnp.where(p0, cand[0][i],
                       jnp.where(p1, cand[1][i],
                                 jnp.where(p2, cand[2][i], cand[3][i])))
             for i in range(4)]
        nrm = jnp.maximum(jnp.sqrt(q[0] * q[0] + q[1] * q[1] + q[2] * q[2] + q[3] * q[3]),
                          1e-12)
        o_ref[...] = jnp.concatenate(
            [q[0] / nrm, q[1] / nrm, q[2] / nrm, q[3] / nrm,
             tp[:, 0:1], tp[:, 1:2], tp[:, 2:3], jnp.zeros_like(vx)], axis=-1)

    def _w(a):
        return pl.BlockSpec(a.shape, lambda i: (0,) * a.ndim)

    tw2 = jnp.pad(p['t_W2'], ((0, 0), (0, 5)))
    tb2 = jnp.pad(p['t_b2'], (0, 5)).reshape(1, -1)
    rw2 = jnp.pad(p['r_W2'], ((0, 0), (0, 5)))
    rb2 = jnp.pad(p['r_b2'], (0, 5)).reshape(1, -1)
    ws = [p['t_W1'], p['t_b1'].reshape(1, -1), tw2, tb2,
          p['r_W1'], p['r_b1'].reshape(1, -1), rw2, rb2]
    return pl.pallas_call(
        body, grid=(_NB,),
        in_specs=[pl.BlockSpec((_RB, 192), lambda i: (i, 0)),
                  pl.BlockSpec((_RB, 192), lambda i: (i, 0))] + [_w(a) for a in ws],
        out_specs=pl.BlockSpec((_RB, 8), lambda i: (i, 0)),
        out_shape=jax.ShapeDtypeStruct((_NP, 8), jnp.float32),
    )(feats, comb, *ws)


def _gnn_layer(x, srcp, dstp, layer, D, want_alpha):
    Wc = 2 * D
    qf = _proj(x, layer['Wq'], D)
    kf = _proj(x, layer['Wk'], D)
    vf = _proj(x, layer['Wv'], D)
    skip = _mm(x, layer['Wskip'])

    outs = _edge_kernel(D, want_alpha)(qf, kf, vf, srcp, dstp)
    agg4 = outs[0].reshape(4, _NP, Wc)
    x_out = _combine(agg4, skip, not want_alpha)
    alpha = None
    if want_alpha:
        ac = outs[5]
        alpha = ac.transpose(2, 3, 5, 0, 1, 4).reshape(_EPAD, 8)[:_E]
    return x_out, alpha


def _vec2skew(v):
    z = jnp.zeros_like(v[..., 0])
    return jnp.stack([
        jnp.stack([z, -v[..., 2], v[..., 1]], axis=-1),
        jnp.stack([v[..., 2], z, -v[..., 0]], axis=-1),
        jnp.stack([-v[..., 1], v[..., 0], z], axis=-1),
    ], axis=-2)


def _skew_to_rmat(v):
    K = _vec2skew(v)
    theta2 = jnp.sum(v * v, axis=-1)[..., None, None]
    theta = jnp.sqrt(theta2 + 1e-20)
    small = theta < 1e-4
    safe_t = jnp.where(small, jnp.ones_like(theta), theta)
    safe_t2 = jnp.where(small, jnp.ones_like(theta2), theta2)
    a = jnp.where(small, 1.0 - theta2 / 6.0, jnp.sin(theta) / safe_t)
    b = jnp.where(small, 0.5 - theta2 / 24.0, (1.0 - jnp.cos(theta)) / safe_t2)
    I = jnp.eye(3, dtype=v.dtype)
    return I + a * K + b * (K @ K)


def _sqrt_pos(x):
    return jnp.where(x > 0, jnp.sqrt(jnp.maximum(x, 1e-20)), jnp.zeros_like(x))


def _matrix_to_quaternion(matrix):
    m00 = matrix[..., 0, 0]; m01 = matrix[..., 0, 1]; m02 = matrix[..., 0, 2]
    m10 = matrix[..., 1, 0]; m11 = matrix[..., 1, 1]; m12 = matrix[..., 1, 2]
    m20 = matrix[..., 2, 0]; m21 = matrix[..., 2, 1]; m22 = matrix[..., 2, 2]
    q_abs = _sqrt_pos(jnp.stack([
        1.0 + m00 + m11 + m22, 1.0 + m00 - m11 - m22,
        1.0 - m00 + m11 - m22, 1.0 - m00 - m11 + m22], axis=-1))
    quat_by_rijk = jnp.stack([
        jnp.stack([q_abs[..., 0] ** 2, m21 - m12, m02 - m20, m10 - m01], axis=-1),
        jnp.stack([m21 - m12, q_abs[..., 1] ** 2, m10 + m01, m02 + m20], axis=-1),
        jnp.stack([m02 - m20, m10 + m01, q_abs[..., 2] ** 2, m12 + m21], axis=-1),
        jnp.stack([m10 - m01, m20 + m02, m21 + m12, q_abs[..., 3] ** 2], axis=-1),
    ], axis=-2)
    quat_candidates = quat_by_rijk / (2.0 * jnp.maximum(q_abs[..., None], 0.1))
    idx = jnp.argmax(q_abs, axis=-1)
    one_hot = jax.nn.one_hot(idx, 4, dtype=matrix.dtype)
    return jnp.sum(quat_candidates * one_hot[..., :, None], axis=-2)


def kernel(xy_pos, time, pcd, edge_index, batch, params):
    p = params
    pcd8 = jnp.pad(pcd, ((0, _NP - _N), (0, 0), (0, 5))).reshape(_NP * 20, 8)
    xy8 = jnp.pad(xy_pos, ((0, _NP - _N), (0, 1)))
    timep = jnp.pad(time.astype(jnp.int32), (0, _NP - _N))
    src = edge_index[0].astype(jnp.int32)
    dst = edge_index[1].astype(jnp.int32)
    srcp = jnp.concatenate([src, jnp.zeros((_EPAD - _E,), jnp.int32)])
    dstp = jnp.concatenate([dst, jnp.zeros((_EPAD - _E,), jnp.int32)])

    tf = _time_kernel()(timep, p['time_emb'])
    combined = _pre_kernel(pcd8, xy8, tf, p)

    x = combined
    alpha = None
    for i, (din, H, D) in enumerate(_LAYER_DIMS):
        last = i == len(_LAYER_DIMS) - 1
        x, alpha = _gnn_layer(x, srcp, dstp, p['gnn'][i], D, last)

    out8 = _heads_kernel(x, combined, p)
    return out8[:_N, :7], alpha
